# baseline (device time: 182709 ns/iter reference)
import jax
import jax.numpy as jnp
from jax import lax
from jax.experimental import pallas as pl
from jax.experimental.pallas import tpu as pltpu

N_DEV = 16
B = 2
SQL = 128
E = 512
HQ = 64
DH = 64
SKV = 128
HCH = 4
QCH = HCH * DH


def kernel(x, Wq, K_ext, V_ext, Wo):
    def body(x_ref, wq_ref, k_ref, v_ref, wo_ref, out_ref,
             wq_g, wo_g, k_bf, v_bf,
             sq_send, sq_recv, so_send, so_recv):
        my = lax.axis_index("i")
        left = lax.rem(my - 1 + N_DEV, N_DEV)
        right = lax.rem(my + 1, N_DEV)

        barrier = pltpu.get_barrier_semaphore()
        for nbr in (left, right):
            pl.semaphore_signal(barrier, inc=1, device_id=(nbr,),
                                device_id_type=pl.DeviceIdType.MESH)
        pl.semaphore_wait(barrier, 2)

        wq_g[my] = wq_ref[...].astype(jnp.bfloat16)
        wo_g[my] = wo_ref[...].astype(jnp.bfloat16)

        for h in range(N_DEV - 1):
            s = lax.rem(my - h + N_DEV, N_DEV)
            rq = pltpu.make_async_remote_copy(
                src_ref=wq_g.at[s], dst_ref=wq_g.at[s],
                send_sem=sq_send.at[h], recv_sem=sq_recv.at[h],
                device_id=(right,), device_id_type=pl.DeviceIdType.MESH)
            ro = pltpu.make_async_remote_copy(
                src_ref=wo_g.at[s], dst_ref=wo_g.at[s],
                send_sem=so_send.at[h], recv_sem=so_recv.at[h],
                device_id=(right,), device_id_type=pl.DeviceIdType.MESH)
            rq.start()
            ro.start()
            rq.wait()
            ro.wait()

        k_bf[...] = k_ref[...].astype(jnp.bfloat16)
        v_bf[...] = v_ref[...].astype(jnp.bfloat16)

        rows = lax.broadcasted_iota(jnp.int32, (SQL, SKV), 0)
        cols = lax.broadcasted_iota(jnp.int32, (SQL, SKV), 1)
        qb = my * 2 + rows // 64
        kb = cols // 64
        mask = (qb == kb) | (lax.rem(qb, 4) == lax.rem(kb, 4))
        row_keep = jnp.max(mask.astype(jnp.float32), axis=1, keepdims=True) > 0.0

        x_bf = x_ref[...].astype(jnp.bfloat16)
        for b in range(B):
            xb = x_bf[b]
            acc = jnp.zeros((SQL, E), jnp.float32)
            for j in range(N_DEV):
                qj = lax.dot_general(
                    xb, wq_g[j], (((1,), (0,)), ((), ())),
                    preferred_element_type=jnp.float32,
                ).astype(jnp.bfloat16)
                ctx_parts = []
                for hh in range(HCH):
                    hg = j * HCH + hh
                    qh = qj[:, hh * DH:(hh + 1) * DH]
                    kh = k_bf[b, :, hg, :]
                    vh = v_bf[b, :, hg, :]
                    s = lax.dot_general(
                        qh, kh, (((1,), (1,)), ((), ())),
                        preferred_element_type=jnp.float32,
                    ) * 0.125
                    s = jnp.where(mask, s, -1e9)
                    m = jnp.max(s, axis=1, keepdims=True)
                    w = jnp.exp(s - m)
                    den = jnp.sum(w, axis=1, keepdims=True)
                    den = jnp.where(row_keep, den, 1.0)
                    w = jnp.where(row_keep, w / den, 0.0)
                    ctx_parts.append(lax.dot_general(
                        w.astype(jnp.bfloat16), vh, (((1,), (0,)), ((), ())),
                        preferred_element_type=jnp.float32,
                    ).astype(jnp.bfloat16))
                ctx_j = jnp.concatenate(ctx_parts, axis=1)
                acc = acc + lax.dot_general(
                    ctx_j, wo_g[j], (((1,), (0,)), ((), ())),
                    preferred_element_type=jnp.float32,
                )
            out_ref[b] = acc

    return pl.pallas_call(
        body,
        out_shape=jax.ShapeDtypeStruct((B, SQL, E), jnp.float32),
        in_specs=[pl.BlockSpec(memory_space=pltpu.VMEM)] * 5,
        out_specs=pl.BlockSpec(memory_space=pltpu.VMEM),
        scratch_shapes=[
            pltpu.VMEM((N_DEV, E, QCH), jnp.bfloat16),
            pltpu.VMEM((N_DEV, QCH, E), jnp.bfloat16),
            pltpu.VMEM((B, SKV, HQ, DH), jnp.bfloat16),
            pltpu.VMEM((B, SKV, HQ, DH), jnp.bfloat16),
            pltpu.SemaphoreType.DMA((N_DEV - 1,)),
            pltpu.SemaphoreType.DMA((N_DEV - 1,)),
            pltpu.SemaphoreType.DMA((N_DEV - 1,)),
            pltpu.SemaphoreType.DMA((N_DEV - 1,)),
        ],
        compiler_params=pltpu.CompilerParams(collective_id=0),
    )(x, Wq, K_ext, V_ext, Wo)


# device time: 161141 ns/iter; 1.1338x vs baseline; 1.1338x over previous
import jax
import jax.numpy as jnp
from jax import lax
from jax.experimental import pallas as pl
from jax.experimental.pallas import tpu as pltpu

N_DEV = 16
B = 2
SQL = 128
E = 512
HQ = 64
DH = 64
SKV = 128
HCH = 4
QCH = HCH * DH


def kernel(x, Wq, K_ext, V_ext, Wo):
    def body(x_ref, wq_ref, k_ref, v_ref, wo_ref, out_ref,
             wq_g, wo_g, k_bf, v_bf,
             sq_send, sq_recv, so_send, so_recv):
        my = lax.axis_index("i")
        left = lax.rem(my - 1 + N_DEV, N_DEV)
        right = lax.rem(my + 1, N_DEV)

        barrier = pltpu.get_barrier_semaphore()
        for nbr in (left, right):
            pl.semaphore_signal(barrier, inc=1, device_id=(nbr,),
                                device_id_type=pl.DeviceIdType.MESH)
        pl.semaphore_wait(barrier, 2)

        wq_g[my] = wq_ref[...].astype(jnp.bfloat16)
        wo_g[my] = wo_ref[...].astype(jnp.bfloat16)

        descs = []
        for h in range(N_DEV - 1):
            sq = lax.rem(my - h + N_DEV, N_DEV)
            so = lax.rem(my + h, N_DEV)
            rq = pltpu.make_async_remote_copy(
                src_ref=wq_g.at[sq], dst_ref=wq_g.at[sq],
                send_sem=sq_send.at[h], recv_sem=sq_recv.at[h],
                device_id=(right,), device_id_type=pl.DeviceIdType.MESH)
            ro = pltpu.make_async_remote_copy(
                src_ref=wo_g.at[so], dst_ref=wo_g.at[so],
                send_sem=so_send.at[h], recv_sem=so_recv.at[h],
                device_id=(left,), device_id_type=pl.DeviceIdType.MESH)
            rq.start()
            ro.start()
            rq.wait()
            ro.wait()

        k_bf[...] = k_ref[...].astype(jnp.bfloat16)
        v_bf[...] = v_ref[...].astype(jnp.bfloat16)

        rows = lax.broadcasted_iota(jnp.int32, (SQL, SKV), 0)
        cols = lax.broadcasted_iota(jnp.int32, (SQL, SKV), 1)
        qb = my * 2 + rows // 64
        kb = cols // 64
        mask = (qb == kb) | (lax.rem(qb, 4) == lax.rem(kb, 4))
        row_keep = jnp.max(mask.astype(jnp.float32), axis=1, keepdims=True) > 0.0

        x_bf = x_ref[...].astype(jnp.bfloat16)
        for b in range(B):
            xb = x_bf[b]
            acc = jnp.zeros((SQL, E), jnp.float32)
            for j in range(N_DEV):
                qj = lax.dot_general(
                    xb, wq_g[j], (((1,), (0,)), ((), ())),
                    preferred_element_type=jnp.float32,
                ).astype(jnp.bfloat16)
                ctx_parts = []
                for hh in range(HCH):
                    hg = j * HCH + hh
                    qh = qj[:, hh * DH:(hh + 1) * DH]
                    kh = k_bf[b, :, hg, :]
                    vh = v_bf[b, :, hg, :]
                    s = lax.dot_general(
                        qh, kh, (((1,), (1,)), ((), ())),
                        preferred_element_type=jnp.float32,
                    ) * 0.125
                    s = jnp.where(mask, s, -1e9)
                    m = jnp.max(s, axis=1, keepdims=True)
                    w = jnp.exp(s - m)
                    den = jnp.sum(w, axis=1, keepdims=True)
                    den = jnp.where(row_keep, den, 1.0)
                    w = jnp.where(row_keep, w / den, 0.0)
                    ctx_parts.append(lax.dot_general(
                        w.astype(jnp.bfloat16), vh, (((1,), (0,)), ((), ())),
                        preferred_element_type=jnp.float32,
                    ).astype(jnp.bfloat16))
                ctx_j = jnp.concatenate(ctx_parts, axis=1)
                acc = acc + lax.dot_general(
                    ctx_j, wo_g[j], (((1,), (0,)), ((), ())),
                    preferred_element_type=jnp.float32,
                )
            out_ref[b] = acc

    return pl.pallas_call(
        body,
        out_shape=jax.ShapeDtypeStruct((B, SQL, E), jnp.float32),
        in_specs=[pl.BlockSpec(memory_space=pltpu.VMEM)] * 5,
        out_specs=pl.BlockSpec(memory_space=pltpu.VMEM),
        scratch_shapes=[
            pltpu.VMEM((N_DEV, E, QCH), jnp.bfloat16),
            pltpu.VMEM((N_DEV, QCH, E), jnp.bfloat16),
            pltpu.VMEM((B, SKV, HQ, DH), jnp.bfloat16),
            pltpu.VMEM((B, SKV, HQ, DH), jnp.bfloat16),
            pltpu.SemaphoreType.DMA((N_DEV - 1,)),
            pltpu.SemaphoreType.DMA((N_DEV - 1,)),
            pltpu.SemaphoreType.DMA((N_DEV - 1,)),
            pltpu.SemaphoreType.DMA((N_DEV - 1,)),
        ],
        compiler_params=pltpu.CompilerParams(collective_id=0),
    )(x, Wq, K_ext, V_ext, Wo)
